# baseline (speedup 1.0000x reference)
import functools

import jax
import jax.numpy as jnp
from jax import lax
from jax.experimental import pallas as pl
from jax.experimental.pallas import tpu as pltpu
from jax.experimental.pallas import tpu_sc as plsc

_NC, _NS, _LANES = 2, 16, 16
_NW = _NC * _NS
_CHUNK = 128
_PAD = 128


def _make_sc_gather(N, D, n_rows):
    zero_row = n_rows - 1
    per_w = N // _NW
    n_chunks = per_w // _CHUNK
    assert n_chunks % 2 == 0

    mesh = plsc.VectorSubcoreMesh(
        core_axis_name="c", subcore_axis_name="s",
        num_cores=_NC, num_subcores=_NS)

    @functools.partial(
        pl.kernel,
        out_type=jax.ShapeDtypeStruct((N, _PAD), jnp.float32),
        mesh=mesh,
        scratch_types=[
            pltpu.VMEM((per_w,), jnp.int32),
            pltpu.VMEM((per_w,), jnp.int32),
            pltpu.VMEM((_CHUNK, _PAD), jnp.float32),
            pltpu.VMEM((_CHUNK, _PAD), jnp.float32),
            pltpu.SemaphoreType.DMA,
            pltpu.SemaphoreType.DMA,
            pltpu.SemaphoreType.DMA,
            pltpu.SemaphoreType.DMA,
        ],
    )
    def sc_gather(foc_hbm, msk_hbm, pe_hbm, fe_hbm,
                  foc_v, idx_v, rows0, rows1, g0, g1, w0, w1):
        wid = lax.axis_index("s") * _NC + lax.axis_index("c")
        base_w = wid * per_w

        pltpu.sync_copy(foc_hbm.at[pl.ds(base_w, per_w)], foc_v)
        pltpu.sync_copy(msk_hbm.at[pl.ds(base_w, per_w)], idx_v)

        def idx_body(j, c):
            sl = pl.ds(j * _LANES, _LANES)
            idx_v[sl] = jnp.where(idx_v[sl] != 0, foc_v[sl], zero_row)
            return c
        lax.fori_loop(0, per_w // _LANES, idx_body, 0, unroll=8)

        rows = (rows0, rows1)
        gsem = (g0, g1)
        wsem = (w0, w1)

        def start_gather(i, b):
            pltpu.async_copy(
                pe_hbm.at[idx_v.at[pl.ds(i * _CHUNK, _CHUNK)]], rows[b],
                gsem[b])

        def wait_gather(b):
            pltpu.make_async_copy(
                pe_hbm.at[idx_v.at[pl.ds(0, _CHUNK)]], rows[b],
                gsem[b]).wait()

        def start_wb(i, b):
            pltpu.async_copy(
                rows[b],
                fe_hbm.at[pl.ds(base_w + i * _CHUNK, _CHUNK), :], wsem[b])

        def wait_wb(b):
            pltpu.make_async_copy(
                rows[b],
                fe_hbm.at[pl.ds(base_w, _CHUNK), :], wsem[b]).wait()

        start_gather(0, 0)

        def pipe_body(i2, c):
            for b in range(2):
                i = i2 * 2 + b
                o = 1 - b
                @pl.when(i >= 1)
                def _():
                    wait_wb(o)
                @pl.when(i + 1 < n_chunks)
                def _():
                    start_gather(i + 1, o)
                wait_gather(b)
                start_wb(i, b)
            return c

        lax.fori_loop(0, n_chunks // 2, pipe_body, 0)
        wait_wb((n_chunks - 1) % 2)

    return sc_gather


def _tc_add(x_flat, fe):
    N, D = x_flat.shape
    blk = 4096
    grid = N // blk

    def add_kernel(x_ref, fe_ref, o_ref):
        o_ref[...] = x_ref[...] + fe_ref[:, 0:D]

    return pl.pallas_call(
        add_kernel,
        grid=(grid,),
        in_specs=[
            pl.BlockSpec((blk, D), lambda i: (i, 0)),
            pl.BlockSpec((blk, _PAD), lambda i: (i, 0)),
        ],
        out_specs=pl.BlockSpec((blk, D), lambda i: (i, 0)),
        out_shape=jax.ShapeDtypeStruct((N, D), jnp.float32),
    )(x_flat, fe)


def kernel(X, focuses, mask, pe):
    B, L, D = X.shape
    N = B * L
    foc = focuses.reshape(N).astype(jnp.int32)
    msk = mask.reshape(N).astype(jnp.int32)
    n_rows = pe.shape[0] + 1
    pe_pad = jnp.zeros((n_rows, _PAD), pe.dtype).at[:pe.shape[0], :D].set(pe)

    fe = _make_sc_gather(N, D, n_rows)(foc, msk, pe_pad)
    out = _tc_add(X.reshape(N, D), fe)
    return out.reshape(B, L, D)


# speedup vs baseline: 17.9032x; 17.9032x over previous
import functools

import jax
import jax.numpy as jnp
from jax import lax
from jax.experimental import pallas as pl
from jax.experimental.pallas import tpu as pltpu
from jax.experimental.pallas import tpu_sc as plsc

_NC, _NS, _LANES = 2, 16, 16
_NW = _NC * _NS
_CHUNK = 128
_PAD = 128


def _make_sc_gather(N, D, n_rows):
    zero_row = n_rows - 1
    per_w = N // _NW
    n_chunks = per_w // _CHUNK
    assert n_chunks % 2 == 0

    mesh = plsc.VectorSubcoreMesh(
        core_axis_name="c", subcore_axis_name="s",
        num_cores=_NC, num_subcores=_NS)

    @functools.partial(
        pl.kernel,
        out_type=jax.ShapeDtypeStruct((N, D), jnp.float32),
        mesh=mesh,
        scratch_types=[
            pltpu.VMEM((per_w,), jnp.int32),
            pltpu.VMEM((per_w,), jnp.int32),
            pltpu.VMEM((_CHUNK, D), jnp.float32),
            pltpu.VMEM((_CHUNK, D), jnp.float32),
            pltpu.VMEM_SHARED((n_rows, D), jnp.float32),
            pltpu.SemaphoreType.DMA,
            pltpu.SemaphoreType.DMA,
            pltpu.SemaphoreType.DMA,
            pltpu.SemaphoreType.DMA,
        ],
    )
    def sc_gather(foc_hbm, msk_hbm, pe_hbm, fe_hbm,
                  foc_v, idx_v, rows0, rows1, pe_sh, g0, g1, w0, w1):
        sid = lax.axis_index("s")
        wid = sid * _NC + lax.axis_index("c")
        base_w = wid * per_w

        @pl.when(sid == 0)
        def _():
            pltpu.sync_copy(pe_hbm, pe_sh)
        plsc.subcore_barrier()

        pltpu.sync_copy(foc_hbm.at[pl.ds(base_w, per_w)], foc_v)
        pltpu.sync_copy(msk_hbm.at[pl.ds(base_w, per_w)], idx_v)

        def idx_body(j, c):
            sl = pl.ds(j * _LANES, _LANES)
            idx_v[sl] = jnp.where(idx_v[sl] != 0, foc_v[sl], zero_row)
            return c
        lax.fori_loop(0, per_w // _LANES, idx_body, 0, unroll=8)

        rows = (rows0, rows1)
        gsem = (g0, g1)
        wsem = (w0, w1)

        def start_gather(i, b):
            pltpu.async_copy(
                pe_sh.at[idx_v.at[pl.ds(i * _CHUNK, _CHUNK)]], rows[b],
                gsem[b])

        def wait_gather(b):
            pltpu.make_async_copy(
                pe_sh.at[idx_v.at[pl.ds(0, _CHUNK)]], rows[b],
                gsem[b]).wait()

        def start_wb(i, b):
            pltpu.async_copy(
                rows[b],
                fe_hbm.at[pl.ds(base_w + i * _CHUNK, _CHUNK), :], wsem[b])

        def wait_wb(b):
            pltpu.make_async_copy(
                rows[b],
                fe_hbm.at[pl.ds(base_w, _CHUNK), :], wsem[b]).wait()

        start_gather(0, 0)

        def pipe_body(i2, c):
            for b in range(2):
                i = i2 * 2 + b
                o = 1 - b
                @pl.when(i >= 1)
                def _():
                    wait_wb(o)
                @pl.when(i + 1 < n_chunks)
                def _():
                    start_gather(i + 1, o)
                wait_gather(b)
                start_wb(i, b)
            return c

        lax.fori_loop(0, n_chunks // 2, pipe_body, 0)
        wait_wb((n_chunks - 1) % 2)

    return sc_gather


def _tc_add(x_flat, fe):
    N, D = x_flat.shape
    blk = 4096
    grid = N // blk

    def add_kernel(x_ref, fe_ref, o_ref):
        o_ref[...] = x_ref[...] + fe_ref[...]

    return pl.pallas_call(
        add_kernel,
        grid=(grid,),
        in_specs=[
            pl.BlockSpec((blk, D), lambda i: (i, 0)),
            pl.BlockSpec((blk, D), lambda i: (i, 0)),
        ],
        out_specs=pl.BlockSpec((blk, D), lambda i: (i, 0)),
        out_shape=jax.ShapeDtypeStruct((N, D), jnp.float32),
    )(x_flat, fe)


def kernel(X, focuses, mask, pe):
    B, L, D = X.shape
    N = B * L
    foc = focuses.reshape(N).astype(jnp.int32)
    msk = mask.reshape(N).astype(jnp.int32)
    n_rows = (pe.shape[0] + 8) // 8 * 8
    pe_pad = jnp.zeros((n_rows, D), pe.dtype).at[:pe.shape[0]].set(pe)

    fe = _make_sc_gather(N, D, n_rows)(foc, msk, pe_pad)
    out = _tc_add(X.reshape(N, D), fe)
    return out.reshape(B, L, D)


# speedup vs baseline: 24.9972x; 1.3962x over previous
import functools

import jax
import jax.numpy as jnp
from jax import lax
from jax.experimental import pallas as pl
from jax.experimental.pallas import tpu as pltpu
from jax.experimental.pallas import tpu_sc as plsc

_NC, _NS, _LANES = 2, 16, 16
_NW = _NC * _NS
_C = 128


def _make_fused(N, D, n_rows):
    zero_row = n_rows - 1
    per_w = N // _NW
    nch = per_w // _C
    nh = nch // 2
    hw = nh * _C
    grp = per_w // 16
    assert nh % 2 == 0 and grp % _LANES == 0 and hw % grp == 0

    mesh = plsc.VectorSubcoreMesh(
        core_axis_name="c", subcore_axis_name="s",
        num_cores=_NC, num_subcores=_NS)

    @functools.partial(
        pl.kernel,
        out_type=jax.ShapeDtypeStruct((N, D), jnp.float32),
        mesh=mesh,
        scratch_types=[
            pltpu.VMEM((grp,), jnp.int32),
            pltpu.VMEM((hw,), jnp.int32),
            pltpu.VMEM((_C, D), jnp.float32),
            pltpu.VMEM((_C, D), jnp.float32),
            pltpu.VMEM((_C, 128), jnp.float32),
            pltpu.VMEM((_C, 128), jnp.float32),
            pltpu.VMEM_SHARED((n_rows, 128), jnp.float32),
            pltpu.SemaphoreType.DMA,
            pltpu.SemaphoreType.DMA,
            pltpu.SemaphoreType.DMA,
            pltpu.SemaphoreType.DMA,
            pltpu.SemaphoreType.DMA,
            pltpu.SemaphoreType.DMA,
        ],
    )
    def fused(foc_hbm, msk_hbm, pe_hbm, x_hbm, out_hbm,
              foc_v, idx_v, x0, x1, r0, r1, pe_sh,
              xs0, xs1, gs0, gs1, ws0, ws1):
        sid = lax.axis_index("s")
        wid = sid * _NC + lax.axis_index("c")
        base_w = wid * per_w

        @pl.when(sid == 0)
        def _():
            pltpu.sync_copy(pe_hbm, pe_sh)
        plsc.subcore_barrier()

        xb = (x0, x1)
        rb = (r0, r1)
        xsem = (xs0, xs1)
        gsem = (gs0, gs1)
        wsem = (ws0, ws1)

        def start_chunk(t_glob, t_loc, buf):
            pltpu.async_copy(
                x_hbm.at[pl.ds(t_glob, _C), :], xb[buf], xsem[buf])
            pltpu.async_copy(
                pe_sh.at[idx_v.at[pl.ds(t_loc, _C)]], rb[buf], gsem[buf])

        def wait_chunk(buf):
            pltpu.make_async_copy(
                x_hbm.at[pl.ds(0, _C), :], xb[buf], xsem[buf]).wait()
            pltpu.make_async_copy(
                pe_sh.at[idx_v.at[pl.ds(0, _C)]], rb[buf], gsem[buf]).wait()

        def start_wb(t_glob, buf):
            pltpu.async_copy(
                xb[buf], out_hbm.at[pl.ds(t_glob, _C), :], wsem[buf])

        def wait_wb(buf):
            pltpu.make_async_copy(
                xb[buf], out_hbm.at[pl.ds(0, _C), :], wsem[buf]).wait()

        def add_chunk(buf):
            x_v, r_v = xb[buf], rb[buf]

            def add_body(t, c):
                for k in range(D // _LANES):
                    sl = pl.ds(k * _LANES, _LANES)
                    plsc.addupdate(x_v.at[t, sl], r_v[t, sl])
                return c
            lax.fori_loop(0, _C, add_body, 0, unroll=4)

        for h in range(2):
            hbase = base_w + h * hw
            pltpu.sync_copy(msk_hbm.at[pl.ds(hbase, hw)], idx_v)
            for g in range(hw // grp):
                pltpu.sync_copy(
                    foc_hbm.at[pl.ds(hbase + g * grp, grp)], foc_v)

                def idx_body(j, c, g=g):
                    sl = pl.ds(g * grp + j * _LANES, _LANES)
                    fsl = pl.ds(j * _LANES, _LANES)
                    idx_v[sl] = jnp.where(idx_v[sl] != 0, foc_v[fsl], zero_row)
                    return c
                lax.fori_loop(0, grp // _LANES, idx_body, 0, unroll=8)

            start_chunk(hbase, 0, 0)

            def pipe_body(c2, c, hbase=hbase):
                for s in range(2):
                    i = c2 * 2 + s
                    o = 1 - s
                    @pl.when(i >= 1)
                    def _():
                        wait_wb(o)
                    @pl.when(i + 1 < nh)
                    def _():
                        t_loc = (i + 1) * _C
                        start_chunk(hbase + t_loc, t_loc, o)
                    wait_chunk(s)
                    add_chunk(s)
                    start_wb(hbase + i * _C, s)
                return c

            lax.fori_loop(0, nh // 2, pipe_body, 0)
            wait_wb((nh - 1) % 2)

    return fused


def kernel(X, focuses, mask, pe):
    B, L, D = X.shape
    N = B * L
    foc = focuses.reshape(N).astype(jnp.int32)
    msk = mask.reshape(N).astype(jnp.int32)
    n_rows = (pe.shape[0] + 8) // 8 * 8
    pe_pad = jnp.zeros((n_rows, 128), pe.dtype).at[:pe.shape[0], :D].set(pe)

    out = _make_fused(N, D, n_rows)(foc, msk, pe_pad, X.reshape(N, D))
    return out.reshape(B, L, D)
